# baseline (speedup 1.0000x reference)
import jax
import jax.numpy as jnp
from jax.experimental import pallas as pl

_B, _N = 4, 512
_IN_NF, _IN_EF, _OUT_F = 16, 8, 16
_TI = 128
_G = 8

_TL = (((0,), (0,)), ((), ()))


def _fused_kernel(wt_ref, a_ref, xt_ref, bd1_ref, bd2_ref, wn_ref, b_ref,
                  wout_ref, xout_ref):
    f32 = jnp.float32
    bf16 = jnp.bfloat16
    bd1 = bd1_ref[...]
    bd2 = bd2_ref[...]
    wn1 = wn_ref[0:16]
    wn2 = wn_ref[16:32]

    r128 = jax.lax.broadcasted_iota(jnp.int32, (128, 16), 0)
    o128 = jax.lax.broadcasted_iota(jnp.int32, (128, 16), 1)
    e1 = jnp.where(r128 % 16 == o128, 1.0, 0.0).astype(f32)
    tr = lambda lhs, rhs: jax.lax.dot_general(
        lhs, rhs, (((1,), (1,)), ((), ())), preferred_element_type=f32)
    be1 = tr(e1, b_ref[0:1, 0:16])
    be2 = tr(e1, b_ref[1:2, 0:16])
    bn1 = tr(e1[0:16], b_ref[2:3, 0:16])
    bn2 = tr(e1[0:16], b_ref[3:4, 0:16])

    xt = xt_ref[0]
    h1 = jnp.maximum(
        jax.lax.dot_general(wn1, xt, _TL, preferred_element_type=f32)
        + bn1, 0.0)
    x1t = jnp.maximum(
        jax.lax.dot_general(wn2, h1, _TL, preferred_element_type=f32)
        + bn2, 0.0)

    wtb = wt_ref[0].astype(bf16)
    hs = []
    for g in range(_TI // _G):
        rhs = wtb[g * _G:(g + 1) * _G].reshape(_G * _IN_EF, _N)
        h = jnp.maximum(
            jax.lax.dot_general(bd1, rhs, _TL, preferred_element_type=f32)
            + be1, 0.0)
        hs.append(h.astype(bf16))
    for g in range(_TI // _G):
        w2 = jnp.maximum(
            jax.lax.dot_general(bd2, hs[g], _TL, preferred_element_type=f32)
            + be2, 0.0)
        wout_ref[0, g * _G:(g + 1) * _G] = w2.reshape(_G, _OUT_F, _N)

    a = a_ref[0]
    asum = jnp.sum(a, axis=1, keepdims=True)
    inv = jnp.where(asum == 0.0, 0.0, 1.0 / asum)
    an = a * inv
    wall = wout_ref[0]
    p = wall * x1t[None] * an[:, None, :]
    xnew = jnp.sum(p, axis=2)
    xout_ref[0] = jnp.transpose(xnew)


@jax.jit
def kernel(A, W, x, We1, be1, We2, be2, Wn1, bn1, Wn2, bn2):
    f32 = jnp.float32
    bf16 = jnp.bfloat16
    wt = jnp.transpose(W, (0, 1, 3, 2))
    xt = jnp.transpose(x, (0, 2, 1))

    bd1 = jnp.zeros((_G * _IN_EF, _G * _OUT_F), bf16)
    bd2 = jnp.zeros((_G * _OUT_F, _G * _OUT_F), bf16)
    we1b = We1[:_IN_EF].astype(bf16)
    we2b = We2.astype(bf16)
    for i in range(_G):
        bd1 = bd1.at[i * _IN_EF:(i + 1) * _IN_EF,
                     i * _OUT_F:(i + 1) * _OUT_F].set(we1b)
        bd2 = bd2.at[i * _OUT_F:(i + 1) * _OUT_F,
                     i * _OUT_F:(i + 1) * _OUT_F].set(we2b)
    wn = jnp.zeros((32, 16), f32)
    wn = wn.at[0:16].set(Wn1).at[16:32].set(Wn2)
    bmat = jnp.zeros((8, 16), f32)
    bmat = (bmat.at[0].set(be1).at[1].set(be2)
                .at[2].set(bn1).at[3].set(bn2))

    const = lambda *shape: pl.BlockSpec(shape, lambda b, i: (0,) * len(shape))
    wout, xout = pl.pallas_call(
        _fused_kernel,
        grid=(_B, _N // _TI),
        in_specs=[
            pl.BlockSpec((1, _TI, _IN_EF, _N), lambda b, i: (b, i, 0, 0)),
            pl.BlockSpec((1, _TI, _N), lambda b, i: (b, i, 0)),
            pl.BlockSpec((1, _IN_NF, _N), lambda b, i: (b, 0, 0)),
            const(_G * _IN_EF, _G * _OUT_F),
            const(_G * _OUT_F, _G * _OUT_F),
            const(32, 16),
            const(8, 16),
        ],
        out_specs=[
            pl.BlockSpec((1, _TI, _OUT_F, _N), lambda b, i: (b, i, 0, 0)),
            pl.BlockSpec((1, _OUT_F, _TI), lambda b, i: (b, 0, i)),
        ],
        out_shape=[
            jax.ShapeDtypeStruct((_B, _N, _OUT_F, _N), f32),
            jax.ShapeDtypeStruct((_B, _OUT_F, _N), f32),
        ],
    )(wt, A, xt, bd1, bd2, wn, bmat)
    return jnp.transpose(wout, (0, 1, 3, 2)), jnp.transpose(xout, (0, 2, 1))
